# baseline (speedup 1.0000x reference)
import functools

import jax
import jax.numpy as jnp
from jax import lax
from jax.experimental import pallas as pl
from jax.experimental.pallas import tpu as pltpu
from jax.experimental.pallas import tpu_sc as plsc

_NC = 2
_NS = 16
_NW = _NC * _NS
_LANES = 16


@functools.lru_cache(maxsize=None)
def _make_kernel(L: int, B: int, V2: int, D: int):
    LG = 8
    CB = 128
    n_lg = L // LG
    n_cb = B // CB
    n_tasks = n_lg * n_cb
    assert n_tasks % _NW == 0
    t_per_w = n_tasks // _NW

    mesh = plsc.VectorSubcoreMesh(core_axis_name="c", subcore_axis_name="s")

    @functools.partial(
        pl.kernel,
        out_type=jax.ShapeDtypeStruct((L, D, B), jnp.float32),
        mesh=mesh,
        scratch_types=[
            pltpu.VMEM((LG, CB), jnp.int32),
            pltpu.VMEM((3, CB), jnp.int32),
            pltpu.VMEM((3, CB, 2 * D), jnp.float32),
            pltpu.VMEM((2, D, CB), jnp.float32),
            pltpu.SemaphoreType.DMA,
            pltpu.SemaphoreType.DMA,
            pltpu.SemaphoreType.DMA,
            pltpu.SemaphoreType.DMA,
            pltpu.SemaphoreType.DMA,
            pltpu.SemaphoreType.DMA,
        ],
        compiler_params=pltpu.CompilerParams(
            use_tc_tiling_on_sc=True, needs_layout_passes=False
        ),
    )
    def embed_kernel(xt_hbm, r2_hbm, ot_hbm, xtb, rid, land, obuf, isem,
                     gs0, gs1, gs2, os0, os1):
        gs = [gs0, gs1, gs2]
        os_ = [os0, os1]
        wid = lax.axis_index("s") * _NC + lax.axis_index("c")
        jvec = lax.iota(jnp.int32, _LANES)
        NG = CB // _LANES

        def fire_gather(l, p):
            for g in range(NG):
                v = xtb[l, pl.ds(g * _LANES, _LANES)]
                rid[p, pl.ds(g * _LANES, _LANES)] = lax.shift_right_logical(v, 1)
            pltpu.async_copy(r2_hbm.at[rid.at[p]], land.at[p], gs[p])

        def wait_gather(p):
            pltpu.make_async_copy(r2_hbm.at[rid.at[p]], land.at[p], gs[p]).wait()

        def transpose_block(l, p, q):
            land_p = land.at[p]
            jvs = [jvec + (g * _LANES) for g in range(NG)]
            odds = [
                lax.bitwise_and(xtb[l, pl.ds(g * _LANES, _LANES)], 1) * D
                for g in range(NG)
            ]

            @plsc.parallel_loop(0, D, 1, unroll=8)
            def dloop(d):
                vals = [
                    plsc.load_gather(land_p, [jvs[g], odds[g] + d])
                    for g in range(NG)
                ]
                for g in range(NG):
                    obuf[q, d, pl.ds(g * _LANES, _LANES)] = vals[g]

        def fire_store(lg, l, cb, q):
            pltpu.async_copy(
                obuf.at[q], ot_hbm.at[lg * LG + l, :, pl.ds(cb * CB, CB)], os_[q]
            )

        def wait_store(lg, l, cb, q):
            pltpu.make_async_copy(
                obuf.at[q], ot_hbm.at[lg * LG + l, :, pl.ds(cb * CB, CB)], os_[q]
            ).wait()

        def do_task(t, carry):
            task = wid * t_per_w + t
            lg = task // n_cb
            cb = task - lg * n_cb
            pltpu.async_copy(
                xt_hbm.at[pl.ds(lg * LG, LG), pl.ds(cb * CB, CB)], xtb, isem
            ).wait()
            fire_gather(0, 0)
            fire_gather(1, 1)
            fire_gather(2, 2)
            for l in range(LG):
                p = l % 3
                q = l % 2
                wait_gather(p)
                if l >= 2:
                    wait_store(lg, l - 2, cb, q)
                transpose_block(l, p, q)
                fire_store(lg, l, cb, q)
                if l + 3 < LG:
                    fire_gather(l + 3, (l + 3) % 3)
            wait_store(lg, LG - 2, cb, 0)
            wait_store(lg, LG - 1, cb, 1)
            return carry

        lax.fori_loop(0, t_per_w, do_task, 0)

    return embed_kernel


def kernel(x, table):
    B, L = x.shape
    V, D = table.shape
    r2 = jnp.concatenate([table[0::2], table[1::2]], axis=1)
    xt = x.T.astype(jnp.int32)
    ot = _make_kernel(L, B, V // 2, D)(xt, r2)
    return ot.transpose(2, 0, 1)


# speedup vs baseline: 6.0969x; 6.0969x over previous
import functools

import jax
import jax.numpy as jnp
from jax import lax
from jax.experimental import pallas as pl
from jax.experimental.pallas import tpu as pltpu
from jax.experimental.pallas import tpu_sc as plsc

_NC = 2
_NS = 16
_NW = _NC * _NS
_LANES = 16


@functools.lru_cache(maxsize=None)
def _make_kernel(L: int, B: int, V2: int, D: int):
    LG = 8
    CB = 128
    n_lg = L // LG
    n_cb = B // CB
    n_tasks = n_lg * n_cb
    assert n_tasks % _NW == 0
    t_per_w = n_tasks // _NW

    mesh = plsc.VectorSubcoreMesh(core_axis_name="c", subcore_axis_name="s")

    @functools.partial(
        pl.kernel,
        out_type=jax.ShapeDtypeStruct((L, D, B), jnp.float32),
        mesh=mesh,
        scratch_types=[
            pltpu.VMEM((LG, CB), jnp.int32),
            pltpu.VMEM((3, CB), jnp.int32),
            pltpu.VMEM((3, CB, 2 * D), jnp.float32),
            pltpu.VMEM((2, D, CB), jnp.float32),
            pltpu.SemaphoreType.DMA,
            pltpu.SemaphoreType.DMA,
            pltpu.SemaphoreType.DMA,
            pltpu.SemaphoreType.DMA,
            pltpu.SemaphoreType.DMA,
            pltpu.SemaphoreType.DMA,
        ],
        compiler_params=pltpu.CompilerParams(
            use_tc_tiling_on_sc=True, needs_layout_passes=False
        ),
    )
    def embed_kernel(xt_hbm, r2_hbm, ot_hbm, xtb, rid, land, obuf, isem,
                     gs0, gs1, gs2, os0, os1):
        gs = [gs0, gs1, gs2]
        os_ = [os0, os1]
        wid = lax.axis_index("s") * _NC + lax.axis_index("c")
        jvec = lax.iota(jnp.int32, _LANES)
        NG = CB // _LANES

        def fire_gather(l, p):
            for g in range(NG):
                v = xtb[l, pl.ds(g * _LANES, _LANES)]
                rid[p, pl.ds(g * _LANES, _LANES)] = lax.shift_right_logical(v, 1)
            pltpu.async_copy(r2_hbm.at[rid.at[p]], land.at[p], gs[p])

        def wait_gather(p):
            pltpu.make_async_copy(r2_hbm.at[rid.at[p]], land.at[p], gs[p]).wait()

        def transpose_block(l, p, q):
            land_p = land.at[p]
            jvs = [jvec + (g * _LANES) for g in range(NG)]
            odds = [
                lax.bitwise_and(xtb[l, pl.ds(g * _LANES, _LANES)], 1) * D
                for g in range(NG)
            ]

            @plsc.parallel_loop(0, D, 1, unroll=8)
            def dloop(d):
                vals = [
                    plsc.load_gather(land_p, [jvs[g], odds[g] + d])
                    for g in range(NG)
                ]
                for g in range(NG):
                    obuf[q, d, pl.ds(g * _LANES, _LANES)] = vals[g]

        def fire_store(lg, l, cb, q):
            pltpu.async_copy(
                obuf.at[q], ot_hbm.at[lg * LG + l, :, pl.ds(cb * CB, CB)], os_[q]
            )

        def wait_store(lg, l, cb, q):
            pltpu.make_async_copy(
                obuf.at[q], ot_hbm.at[lg * LG + l, :, pl.ds(cb * CB, CB)], os_[q]
            ).wait()

        def do_task(t, carry):
            task = wid * t_per_w + t
            lg = task // n_cb
            cb = task - lg * n_cb
            pltpu.async_copy(
                xt_hbm.at[pl.ds(lg * LG, LG), pl.ds(cb * CB, CB)], xtb, isem
            ).wait()
            fire_gather(0, 0)
            fire_gather(1, 1)
            fire_gather(2, 2)
            for l in range(LG):
                p = l % 3
                q = l % 2
                wait_gather(p)
                if l >= 2:
                    wait_store(lg, l - 2, cb, q)
                transpose_block(l, p, q)
                fire_store(lg, l, cb, q)
                if l + 3 < LG:
                    fire_gather(l + 3, (l + 3) % 3)
            wait_store(lg, LG - 2, cb, 0)
            wait_store(lg, LG - 1, cb, 1)
            return carry

        lax.fori_loop(0, t_per_w, do_task, 0)

    return embed_kernel


def kernel(x, table):
    B, L = x.shape
    V, D = table.shape
    r2 = table.reshape(V // 2, 2 * D)
    xt = x.T.astype(jnp.int32)
    ot = _make_kernel(L, B, V // 2, D)(xt, r2)
    return ot.transpose(2, 0, 1)


# speedup vs baseline: 7.8491x; 1.2874x over previous
import functools

import jax
import jax.numpy as jnp
from jax import lax
from jax.experimental import pallas as pl
from jax.experimental.pallas import tpu as pltpu
from jax.experimental.pallas import tpu_sc as plsc

_NC = 2
_NS = 16
_NW = _NC * _NS

_CHUNK = 640
_NBUF = 2


@functools.lru_cache(maxsize=None)
def _make_gather(B: int, D: int):
    assert B % _NW == 0
    b_per_w = B // _NW
    chunk = _CHUNK
    assert b_per_w % (chunk * _NBUF) == 0
    n_chunks = b_per_w // chunk

    mesh = plsc.VectorSubcoreMesh(core_axis_name="c", subcore_axis_name="s")

    @functools.partial(
        pl.kernel,
        out_type=jax.ShapeDtypeStruct((B, D), jnp.float32),
        mesh=mesh,
        scratch_types=[
            pltpu.VMEM((_NBUF, chunk), jnp.int32),
            pltpu.VMEM((_NBUF, chunk, D), jnp.float32),
            pltpu.SemaphoreType.DMA,
            pltpu.SemaphoreType.DMA,
            pltpu.SemaphoreType.DMA,
            pltpu.SemaphoreType.DMA,
        ],
        compiler_params=pltpu.CompilerParams(use_tc_tiling_on_sc=False),
    )
    def gather_kernel(x_hbm, table_hbm, out_hbm, idx_v, rows_v, gs0, gs1, os0, os1):
        gs = [gs0, gs1]
        os_ = [os0, os1]
        wid = lax.axis_index("s") * _NC + lax.axis_index("c")
        base = wid * b_per_w

        def fire(i, b):
            off = base + i * chunk
            pltpu.sync_copy(x_hbm.at[pl.ds(off, chunk)], idx_v.at[b])
            pltpu.async_copy(table_hbm.at[idx_v.at[b]], rows_v.at[b], gs[b])

        def wait_gathers(b):
            pltpu.make_async_copy(
                table_hbm.at[idx_v.at[b]], rows_v.at[b], gs[b]
            ).wait()

        def store(i, b):
            off = base + i * chunk
            pltpu.async_copy(rows_v.at[b], out_hbm.at[pl.ds(off, chunk)], os_[b])

        def wait_store(i, b):
            off = base + i * chunk
            pltpu.make_async_copy(
                rows_v.at[b], out_hbm.at[pl.ds(off, chunk)], os_[b]
            ).wait()

        fire(0, 0)
        fire(1, 1)

        def group(g, carry):
            for b in range(_NBUF):
                i = 2 * g + b + 2
                wait_gathers(b)
                store(i - 2, b)
                wait_store(i - 2, b)
                fire(i, b)
            return carry

        lax.fori_loop(0, (n_chunks - 2) // _NBUF, group, 0)

        for b in range(_NBUF):
            i = n_chunks - 2 + b
            wait_gathers(b)
            store(i, b)
        for b in range(_NBUF):
            wait_store(n_chunks - 2 + b, b)

    return gather_kernel


def kernel(x, table):
    B = x.shape[0] * x.shape[1]
    D = table.shape[1]
    xf = x.reshape(B).astype(jnp.int32)
    out = _make_gather(B, D)(xf, table)
    return out.reshape(x.shape[0], x.shape[1], D)
